# baseline (speedup 1.0000x reference)
import jax
import jax.numpy as jnp
from jax.experimental import pallas as pl
from jax.experimental.pallas import tpu as pltpu

N = 512
D = 64
R = 32
RI = 2
BI = 32
NEG = -1e30


def _gcn_block(cc_ref, ct_ref, s0_ref, s1_ref, nei_ref, h_ref, ht_ref,
               c_ref, og_ref,
               relw_ref, relb_ref, ngw_ref, ngb_ref, war_ref, warhjt_ref,
               rels_ref, wars_ref, wnei_ref, wneib_ref,
               hout_ref, cout_ref):
    i = pl.program_id(0)
    bf16 = jnp.bfloat16
    f32 = jnp.float32

    cc = cc_ref[...].reshape(BI * N, RI).astype(bf16)
    r = jnp.dot(cc, relw_ref[...].astype(bf16), preferred_element_type=f32)
    r = jnp.maximum(r + relb_ref[...], 0.0).astype(bf16)
    ngw = ngw_ref[...].astype(bf16)
    glog = jnp.dot(r, ngw[:R], preferred_element_type=f32)

    h_all = h_ref[...].astype(bf16)
    h_blk = h_ref[pl.ds(i * BI, BI), :]
    a_i = jnp.dot(h_blk.astype(bf16), ngw[R:R + D],
                  preferred_element_type=f32)
    a_i = a_i + ngb_ref[...]
    b_j = jnp.dot(h_all, ngw[R + D:], preferred_element_type=f32)

    lg = (glog.reshape(BI, N, D) + a_i[:, None, :] + b_j[None, :, :])
    gate = jax.nn.sigmoid(lg.astype(bf16))
    q = gate * h_all[None, :, :]

    cil = ct_ref[...].astype(bf16)
    c0 = jnp.dot(cil, s0_ref[...], preferred_element_type=f32)
    c1 = jnp.dot(cil, s1_ref[...], preferred_element_type=f32)
    t = jnp.zeros((BI, N), f32)
    for k in range(R):
        rk = jnp.maximum(c0 * rels_ref[0, k] + c1 * rels_ref[1, k]
                         + rels_ref[2, k], 0.0)
        t = t + rk * wars_ref[0, k]
    aw = jnp.dot(h_blk, war_ref[R:R + D], preferred_element_type=f32)
    bw = jnp.dot(warhjt_ref[...], ht_ref[...], preferred_element_type=f32)
    tt = t + aw + bw + wars_ref[1, 0]

    m2 = (nei_ref[...] > 0) & (tt != 0.0)
    mx = jnp.max(jnp.where(m2, tt, NEG), axis=1, keepdims=True)
    w = jnp.where(m2, jnp.exp(tt - mx), 0.0)
    s = jnp.sum(w, axis=1, keepdims=True)
    p = (w / jnp.where(s > 0.0, s, 1.0)).astype(bf16)

    h_sum = jax.lax.dot_general(
        p, q, (((1,), (1,)), ((0,), (0,))),
        preferred_element_type=f32)

    c_out = jnp.dot(h_sum, wnei_ref[...], preferred_element_type=f32)
    c_out = c_out + wneib_ref[...] + c_ref[...]
    cout_ref[...] = c_out
    hout_ref[...] = og_ref[...] * jnp.tanh(c_out)


def kernel(corr_index, nei_index, nei_num, outgate, self_h, self_c,
           rel_W, rel_b, ngate_W, ngate_b, war_W, war_b, wnei_W, wnei_b):
    n = corr_index.shape[0]
    d = self_h.shape[1]
    ri = corr_index.shape[2]
    r = rel_W.shape[1]
    assert (n, d, ri, r) == (N, D, RI, R)
    f32 = jnp.float32

    cint = corr_index.reshape(n, n * ri)
    bf = jnp.bfloat16
    eye = jnp.eye(n, dtype=bf)
    zro = jnp.zeros((n, n), bf)
    s0 = jnp.stack([eye, zro], axis=1).reshape(n * ri, n)
    s1 = jnp.stack([zro, eye], axis=1).reshape(n * ri, n)

    rels = jnp.stack([rel_W[0], rel_W[1], rel_b])
    wars = jnp.zeros((2, r), f32).at[0].set(war_W[:r, 0]).at[1, 0].set(war_b[0])

    grid = (n // BI,)
    full = lambda shape: pl.BlockSpec(shape, lambda i: (0,) * len(shape))
    row_blk = lambda shape: pl.BlockSpec(shape, lambda i: (i,) + (0,) * (len(shape) - 1))
    ct_blk = pl.BlockSpec((BI, n * ri), lambda i: (i, 0))
    smem = lambda shape: pl.BlockSpec(shape, lambda i: (0,) * len(shape),
                                      memory_space=pltpu.SMEM)

    h_out, c_out = pl.pallas_call(
        _gcn_block,
        grid=grid,
        in_specs=[
            row_blk((BI, n, ri)),
            ct_blk,
            full((n * ri, n)),
            full((n * ri, n)),
            row_blk((BI, n)),
            full((n, d)),
            full((d, n)),
            row_blk((BI, d)),
            row_blk((BI, d)),
            full((ri, r)),
            full((1, r)),
            full((r + 2 * d, d)),
            full((1, d)),
            full((r + 2 * d, 1)),
            full((1, d)),
            smem((3, r)),
            smem((2, r)),
            full((d, d)),
            full((1, d)),
        ],
        out_specs=[row_blk((BI, d)), row_blk((BI, d))],
        out_shape=[
            jax.ShapeDtypeStruct((n, d), f32),
            jax.ShapeDtypeStruct((n, d), f32),
        ],
        compiler_params=pltpu.CompilerParams(
            dimension_semantics=("arbitrary",),
        ),
    )(corr_index, cint, s0, s1, nei_index, self_h, self_h.T, self_c, outgate,
      rel_W, rel_b.reshape(1, r), ngate_W, ngate_b.reshape(1, d),
      war_W, war_W[r + d:].T, rels, wars, wnei_W, wnei_b.reshape(1, d))

    return (outgate, h_out, c_out)


# speedup vs baseline: 1.0381x; 1.0381x over previous
import jax
import jax.numpy as jnp
import numpy as np
from jax.experimental import pallas as pl
from jax.experimental.pallas import tpu as pltpu

N = 512
D = 64
R = 32
RI = 2
BI = 32
NEG = -1e30


def _gcn_block(cc_ref, s0_ref, s1_ref, nei_ref, h_ref, ht_ref,
               c_ref, og_ref,
               relw_ref, relb_ref, ngw_ref, ngb_ref, war_ref, warhjt_ref,
               rels_ref, wars_ref, wnei_ref, wneib_ref,
               hout_ref, cout_ref):
    i = pl.program_id(0)
    bf16 = jnp.bfloat16
    f32 = jnp.float32

    cc = cc_ref[...].reshape(BI * N, RI).astype(bf16)
    r = jnp.dot(cc, relw_ref[...].astype(bf16), preferred_element_type=f32)
    r = jnp.maximum(r + relb_ref[...], 0.0).astype(bf16)
    ngw = ngw_ref[...].astype(bf16)
    glog = jnp.dot(r, ngw[:R], preferred_element_type=f32)

    h_all = h_ref[...].astype(bf16)
    h_blk = h_ref[pl.ds(i * BI, BI), :]
    a_i = jnp.dot(h_blk.astype(bf16), ngw[R:R + D],
                  preferred_element_type=f32)
    a_i = a_i + ngb_ref[...]
    b_j = jnp.dot(h_all, ngw[R + D:], preferred_element_type=f32)

    lg = (glog.reshape(BI, N, D) + a_i[:, None, :] + b_j[None, :, :])
    gate = jax.nn.sigmoid(lg.astype(bf16))
    q = gate * h_all[None, :, :]

    cil = cc_ref[...].astype(bf16).reshape(BI, RI * N)
    c0 = jnp.dot(cil, s0_ref[...], preferred_element_type=f32)
    c1 = jnp.dot(cil, s1_ref[...], preferred_element_type=f32)
    t = jnp.zeros((BI, N), f32)
    for k in range(R):
        rk = jnp.maximum(c0 * rels_ref[0, k] + c1 * rels_ref[1, k]
                         + rels_ref[2, k], 0.0)
        t = t + rk * wars_ref[0, k]
    aw = jnp.dot(h_blk, war_ref[R:R + D], preferred_element_type=f32)
    bw = jnp.dot(warhjt_ref[...], ht_ref[...], preferred_element_type=f32)
    tt = t + aw + bw + wars_ref[1, 0]

    m2 = (nei_ref[...] > 0) & (tt != 0.0)
    mx = jnp.max(jnp.where(m2, tt, NEG), axis=1, keepdims=True)
    w = jnp.where(m2, jnp.exp(tt - mx), 0.0)
    s = jnp.sum(w, axis=1, keepdims=True)
    p = (w / jnp.where(s > 0.0, s, 1.0)).astype(bf16)

    h_sum = jax.lax.dot_general(
        p, q, (((1,), (1,)), ((0,), (0,))),
        preferred_element_type=f32)

    c_out = jnp.dot(h_sum, wnei_ref[...], preferred_element_type=f32)
    c_out = c_out + wneib_ref[...] + c_ref[...]
    cout_ref[...] = c_out
    hout_ref[...] = og_ref[...] * jnp.tanh(c_out)


def kernel(corr_index, nei_index, nei_num, outgate, self_h, self_c,
           rel_W, rel_b, ngate_W, ngate_b, war_W, war_b, wnei_W, wnei_b):
    n = corr_index.shape[0]
    d = self_h.shape[1]
    ri = corr_index.shape[2]
    r = rel_W.shape[1]
    assert (n, d, ri, r) == (N, D, RI, R)
    f32 = jnp.float32

    s_np = np.zeros((ri, n * ri, n), np.float32)
    for c in range(ri):
        s_np[c, np.arange(n) * ri + c, np.arange(n)] = 1.0
    s0 = jnp.asarray(s_np[0], dtype=jnp.bfloat16)
    s1 = jnp.asarray(s_np[1], dtype=jnp.bfloat16)

    rels = jnp.stack([rel_W[0], rel_W[1], rel_b])
    wars = jnp.zeros((2, r), f32).at[0].set(war_W[:r, 0]).at[1, 0].set(war_b[0])

    grid = (n // BI,)
    full = lambda shape: pl.BlockSpec(shape, lambda i: (0,) * len(shape))
    row_blk = lambda shape: pl.BlockSpec(shape, lambda i: (i,) + (0,) * (len(shape) - 1))
    smem = lambda shape: pl.BlockSpec(shape, lambda i: (0,) * len(shape),
                                      memory_space=pltpu.SMEM)

    h_out, c_out = pl.pallas_call(
        _gcn_block,
        grid=grid,
        in_specs=[
            row_blk((BI, n, ri)),
            full((n * ri, n)),
            full((n * ri, n)),
            row_blk((BI, n)),
            full((n, d)),
            full((d, n)),
            row_blk((BI, d)),
            row_blk((BI, d)),
            full((ri, r)),
            full((1, r)),
            full((r + 2 * d, d)),
            full((1, d)),
            full((r + 2 * d, 1)),
            full((1, d)),
            smem((3, r)),
            smem((2, r)),
            full((d, d)),
            full((1, d)),
        ],
        out_specs=[row_blk((BI, d)), row_blk((BI, d))],
        out_shape=[
            jax.ShapeDtypeStruct((n, d), f32),
            jax.ShapeDtypeStruct((n, d), f32),
        ],
        compiler_params=pltpu.CompilerParams(
            dimension_semantics=("parallel",),
        ),
    )(corr_index, s0, s1, nei_index, self_h, self_h.T, self_c, outgate,
      rel_W, rel_b.reshape(1, r), ngate_W, ngate_b.reshape(1, d),
      war_W, war_W[r + d:].T, rels, wars, wnei_W, wnei_b.reshape(1, d))

    return (outgate, h_out, c_out)


# speedup vs baseline: 1.5983x; 1.5396x over previous
import jax
import jax.numpy as jnp
import numpy as np
from jax.experimental import pallas as pl
from jax.experimental.pallas import tpu as pltpu

N = 512
D = 64
R = 32
RI = 2
BI = 32
NEG = -1e30


def _gcn_block(ct_ref, nei_ref, h_ref, ht_ref,
               c_ref, og_ref,
               relw_ref, relb_ref, ngw_ref, ngb_ref, war_ref, warhjt_ref,
               rels_ref, wars_ref, wnei_ref, wneib_ref,
               hout_ref, cout_ref):
    i = pl.program_id(0)
    bf16 = jnp.bfloat16
    f32 = jnp.float32

    cc = jnp.swapaxes(ct_ref[...].astype(bf16), 1, 2).reshape(BI * N, RI)
    r = jnp.dot(cc, relw_ref[...].astype(bf16), preferred_element_type=f32)
    r = jnp.maximum(r + relb_ref[...], 0.0).astype(bf16)
    ngw = ngw_ref[...].astype(bf16)
    glog = jnp.dot(r, ngw[:R], preferred_element_type=f32)

    h_all = h_ref[...].astype(bf16)
    h_blk = h_ref[pl.ds(i * BI, BI), :]
    a_i = jnp.dot(h_blk.astype(bf16), ngw[R:R + D],
                  preferred_element_type=f32)
    a_i = a_i + ngb_ref[...]
    b_j = jnp.dot(h_all, ngw[R + D:], preferred_element_type=f32)

    lg = (glog.reshape(BI, N, D) + a_i[:, None, :] + b_j[None, :, :])
    gate = jax.nn.sigmoid(lg.astype(bf16))
    q = gate * h_all[None, :, :]

    c0 = ct_ref[:, 0, :]
    c1 = ct_ref[:, 1, :]
    t = jnp.zeros((BI, N), f32)
    for k in range(R):
        rk = jnp.maximum(c0 * rels_ref[0, k] + c1 * rels_ref[1, k]
                         + rels_ref[2, k], 0.0)
        t = t + rk * wars_ref[0, k]
    aw = jnp.dot(h_blk, war_ref[R:R + D], preferred_element_type=f32)
    bw = jnp.dot(warhjt_ref[...], ht_ref[...], preferred_element_type=f32)
    tt = t + aw + bw + wars_ref[1, 0]

    m2 = (nei_ref[...] > 0) & (tt != 0.0)
    mx = jnp.max(jnp.where(m2, tt, NEG), axis=1, keepdims=True)
    w = jnp.where(m2, jnp.exp(tt - mx), 0.0)
    s = jnp.sum(w, axis=1, keepdims=True)
    p = (w / jnp.where(s > 0.0, s, 1.0)).astype(bf16)

    h_sum = jax.lax.dot_general(
        p, q, (((1,), (1,)), ((0,), (0,))),
        preferred_element_type=f32)

    c_out = jnp.dot(h_sum, wnei_ref[...], preferred_element_type=f32)
    c_out = c_out + wneib_ref[...] + c_ref[...]
    cout_ref[...] = c_out
    hout_ref[...] = og_ref[...] * jnp.tanh(c_out)


def kernel(corr_index, nei_index, nei_num, outgate, self_h, self_c,
           rel_W, rel_b, ngate_W, ngate_b, war_W, war_b, wnei_W, wnei_b):
    n = corr_index.shape[0]
    d = self_h.shape[1]
    ri = corr_index.shape[2]
    r = rel_W.shape[1]
    assert (n, d, ri, r) == (N, D, RI, R)
    f32 = jnp.float32

    ct2 = jnp.transpose(corr_index, (0, 2, 1))

    rels = jnp.stack([rel_W[0], rel_W[1], rel_b])
    wars = jnp.zeros((2, r), f32).at[0].set(war_W[:r, 0]).at[1, 0].set(war_b[0])

    grid = (n // BI,)
    full = lambda shape: pl.BlockSpec(shape, lambda i: (0,) * len(shape))
    row_blk = lambda shape: pl.BlockSpec(shape, lambda i: (i,) + (0,) * (len(shape) - 1))
    smem = lambda shape: pl.BlockSpec(shape, lambda i: (0,) * len(shape),
                                      memory_space=pltpu.SMEM)

    h_out, c_out = pl.pallas_call(
        _gcn_block,
        grid=grid,
        in_specs=[
            row_blk((BI, ri, n)),
            row_blk((BI, n)),
            full((n, d)),
            full((d, n)),
            row_blk((BI, d)),
            row_blk((BI, d)),
            full((ri, r)),
            full((1, r)),
            full((r + 2 * d, d)),
            full((1, d)),
            full((r + 2 * d, 1)),
            full((1, d)),
            smem((3, r)),
            smem((2, r)),
            full((d, d)),
            full((1, d)),
        ],
        out_specs=[row_blk((BI, d)), row_blk((BI, d))],
        out_shape=[
            jax.ShapeDtypeStruct((n, d), f32),
            jax.ShapeDtypeStruct((n, d), f32),
        ],
        compiler_params=pltpu.CompilerParams(
            dimension_semantics=("parallel",),
        ),
    )(ct2, nei_index, self_h, self_h.T, self_c, outgate,
      rel_W, rel_b.reshape(1, r), ngate_W, ngate_b.reshape(1, d),
      war_W, war_W[r + d:].T, rels, wars, wnei_W, wnei_b.reshape(1, d))

    return (outgate, h_out, c_out)
